# Initial kernel scaffold; baseline (speedup 1.0000x reference)
import jax
import jax.numpy as jnp
from jax.experimental import pallas as pl


def kernel(idx, targets, table):
    raise NotImplementedError("write your pallas kernel here")


# speedup vs baseline: 1.3146x; 1.3146x over previous
import functools

import jax
import jax.numpy as jnp
from jax import lax
from jax.experimental import pallas as pl
from jax.experimental.pallas import tpu as pltpu
from jax.experimental.pallas import tpu_sc as plsc

VOCAB = 1000
N_TOK = 1024 * 200

NC, NS, L = 2, 16, 16
NW = NC * NS
B_PER_W = N_TOK // NW
CHUNK = 64
N_CHUNKS = B_PER_W // CHUNK


def _lse_body(t_ref, o_ref):
    x = t_ref[...]
    m = jnp.max(x, axis=1, keepdims=True)
    o_ref[...] = jnp.log(jnp.sum(jnp.exp(x - m), axis=1, keepdims=True)) + m


def _row_lse(table):
    return pl.pallas_call(
        _lse_body,
        out_shape=jax.ShapeDtypeStruct((VOCAB, 1), jnp.float32),
    )(table).reshape(VOCAB)


@functools.partial(
    pl.kernel,
    out_type=[
        jax.ShapeDtypeStruct((N_TOK, VOCAB), jnp.float32),
        jax.ShapeDtypeStruct((NW, L), jnp.float32),
    ],
    mesh=plsc.VectorSubcoreMesh(core_axis_name="c", subcore_axis_name="s"),
    compiler_params=pltpu.CompilerParams(
        use_tc_tiling_on_sc=False,
        needs_layout_passes=False,
    ),
    scratch_types=[
        pltpu.VMEM((VOCAB,), jnp.float32),
        pltpu.VMEM((CHUNK,), jnp.int32),
        pltpu.VMEM((CHUNK,), jnp.int32),
        pltpu.VMEM((CHUNK, VOCAB), jnp.float32),
        pltpu.VMEM((L,), jnp.float32),
        pltpu.SemaphoreType.DMA,
    ],
)
def _sc_gather_loss(idx_hbm, tgt_hbm, table_hbm, lse_hbm,
                    out_hbm, part_hbm,
                    lse_v, idx_v, tgt_v, rows_v, acc_v, sem):
    wid = lax.axis_index("s") * NC + lax.axis_index("c")
    base = wid * B_PER_W

    pltpu.sync_copy(lse_hbm, lse_v)
    acc_v[...] = jnp.zeros((L,), jnp.float32)

    def chunk_step(i, carry):
        off = base + i * CHUNK
        pltpu.sync_copy(idx_hbm.at[pl.ds(off, CHUNK)], idx_v)
        pltpu.sync_copy(tgt_hbm.at[pl.ds(off, CHUNK)], tgt_v)
        pltpu.async_copy(table_hbm.at[idx_v], rows_v, sem).wait()
        pltpu.sync_copy(rows_v, out_hbm.at[pl.ds(off, CHUNK)])
        for j in range(CHUNK // L):
            s = pl.ds(j * L, L)
            iv = idx_v[s]
            tv = tgt_v[s]
            rid = lax.iota(jnp.int32, L) + (j * L)
            lse_g = plsc.load_gather(lse_v, [iv])
            tgt_g = plsc.load_gather(rows_v, [rid, tv])
            acc_v[...] = acc_v[...] + (lse_g - tgt_g)
        return carry

    lax.fori_loop(0, N_CHUNKS, chunk_step, 0)
    pltpu.sync_copy(acc_v, part_hbm.at[wid])


def kernel(idx, targets, table):
    idx_f = idx.reshape(N_TOK).astype(jnp.int32)
    tgt_f = targets.reshape(N_TOK).astype(jnp.int32)
    lse = _row_lse(table)
    logits2, parts = _sc_gather_loss(idx_f, tgt_f, table, lse)
    loss = jnp.sum(parts) / jnp.float32(N_TOK)
    return (logits2, loss)


# speedup vs baseline: 1.3988x; 1.0641x over previous
import functools

import jax
import jax.numpy as jnp
from jax import lax
from jax.experimental import pallas as pl
from jax.experimental.pallas import tpu as pltpu
from jax.experimental.pallas import tpu_sc as plsc

VOCAB = 1000
N_TOK = 1024 * 200

NC, NS, L = 2, 16, 16
NW = NC * NS
B_PER_W = N_TOK // NW
CHUNK = 64
N_CHUNKS = B_PER_W // CHUNK


def _lse_body(t_ref, o_ref):
    x = t_ref[...]
    m = jnp.max(x, axis=1, keepdims=True)
    o_ref[...] = jnp.log(jnp.sum(jnp.exp(x - m), axis=1, keepdims=True)) + m


def _row_lse(table):
    return pl.pallas_call(
        _lse_body,
        out_shape=jax.ShapeDtypeStruct((VOCAB, 1), jnp.float32),
    )(table).reshape(VOCAB)


@functools.partial(
    pl.kernel,
    out_type=[
        jax.ShapeDtypeStruct((N_TOK, VOCAB), jnp.float32),
        jax.ShapeDtypeStruct((NW, L), jnp.float32),
    ],
    mesh=plsc.VectorSubcoreMesh(core_axis_name="c", subcore_axis_name="s"),
    compiler_params=pltpu.CompilerParams(
        use_tc_tiling_on_sc=False,
        needs_layout_passes=False,
    ),
    scratch_types=[
        pltpu.VMEM((VOCAB,), jnp.float32),
        pltpu.VMEM((CHUNK,), jnp.int32),
        pltpu.VMEM((CHUNK,), jnp.int32),
        pltpu.VMEM((CHUNK,), jnp.int32),
        pltpu.VMEM((CHUNK,), jnp.int32),
        pltpu.VMEM((CHUNK, VOCAB), jnp.float32),
        pltpu.VMEM((CHUNK, VOCAB), jnp.float32),
        pltpu.VMEM((L,), jnp.float32),
        pltpu.SemaphoreType.DMA,
        pltpu.SemaphoreType.DMA,
        pltpu.SemaphoreType.DMA,
        pltpu.SemaphoreType.DMA,
    ],
)
def _sc_gather_loss(idx_hbm, tgt_hbm, table_hbm, lse_hbm,
                    out_hbm, part_hbm,
                    lse_v, idx0, idx1, tgt0, tgt1, rows0, rows1, acc_v,
                    sg0, sg1, sw0, sw1):
    wid = lax.axis_index("s") * NC + lax.axis_index("c")
    base = wid * B_PER_W
    bufs = ((idx0, tgt0, rows0, sg0, sw0), (idx1, tgt1, rows1, sg1, sw1))

    pltpu.sync_copy(lse_hbm, lse_v)
    acc_v[...] = jnp.zeros((L,), jnp.float32)

    def load_itgt(c, idx_b, tgt_b):
        off = base + c * CHUNK
        pltpu.sync_copy(idx_hbm.at[pl.ds(off, CHUNK)], idx_b)
        pltpu.sync_copy(tgt_hbm.at[pl.ds(off, CHUNK)], tgt_b)

    def loss(idx_b, tgt_b, rows_b):
        for j in range(CHUNK // L):
            s = pl.ds(j * L, L)
            rid = lax.iota(jnp.int32, L) + (j * L)
            lse_g = plsc.load_gather(lse_v, [idx_b[s]])
            tgt_g = plsc.load_gather(rows_b, [rid, tgt_b[s]])
            acc_v[...] = acc_v[...] + (lse_g - tgt_g)

    def visit(c, b, first):
        idx_b, tgt_b, rows_b, sg_b, sw_b = bufs[b]
        idx_o, tgt_o, rows_o, sg_o, sw_o = bufs[1 - b]
        if not first:
            pltpu.make_async_copy(
                rows_o, out_hbm.at[pl.ds(base, CHUNK)], sw_o).wait()

        @pl.when(c + 1 < N_CHUNKS)
        def _():
            pltpu.async_copy(table_hbm.at[idx_o], rows_o, sg_o)

        pltpu.make_async_copy(table_hbm.at[idx_b], rows_b, sg_b).wait()
        loss(idx_b, tgt_b, rows_b)
        pltpu.async_copy(rows_b, out_hbm.at[pl.ds(base + c * CHUNK, CHUNK)],
                         sw_b)

        @pl.when(c + 2 < N_CHUNKS)
        def _():
            load_itgt(c + 2, idx_b, tgt_b)

    load_itgt(0, idx0, tgt0)
    load_itgt(1, idx1, tgt1)
    pltpu.async_copy(table_hbm.at[idx0], rows0, sg0)
    visit(jnp.int32(0), 0, True)
    visit(jnp.int32(1), 1, False)

    def pair(p, carry):
        visit(2 * p, 0, False)
        visit(2 * p + 1, 1, False)
        return carry

    lax.fori_loop(1, N_CHUNKS // 2, pair, 0)
    pltpu.make_async_copy(rows1, out_hbm.at[pl.ds(base, CHUNK)], sw1).wait()
    pltpu.sync_copy(acc_v, part_hbm.at[wid])


def kernel(idx, targets, table):
    idx_f = idx.reshape(N_TOK).astype(jnp.int32)
    tgt_f = targets.reshape(N_TOK).astype(jnp.int32)
    lse = _row_lse(table)
    logits2, parts = _sc_gather_loss(idx_f, tgt_f, table, lse)
    loss = jnp.sum(parts) / jnp.float32(N_TOK)
    return (logits2, loss)


# speedup vs baseline: 2.3874x; 1.7067x over previous
import functools

import jax
import jax.numpy as jnp
from jax import lax
from jax.experimental import pallas as pl
from jax.experimental.pallas import tpu as pltpu
from jax.experimental.pallas import tpu_sc as plsc

VOCAB = 1000
VPAD = 1024
N_TOK = 1024 * 200

NC, NS, L = 2, 16, 16
NW = NC * NS
B_PER_W = N_TOK // NW
CHUNK = 32
N_CHUNKS = B_PER_W // CHUNK


def _lse_body(t_ref, o_ref):
    x = t_ref[...]
    m = jnp.max(x, axis=1, keepdims=True)
    o_ref[...] = jnp.log(jnp.sum(jnp.exp(x - m), axis=1, keepdims=True)) + m


def _row_lse(table):
    return pl.pallas_call(
        _lse_body,
        out_shape=jax.ShapeDtypeStruct((VOCAB, 1), jnp.float32),
    )(table).reshape(VOCAB)


@functools.partial(
    pl.kernel,
    out_type=[
        jax.ShapeDtypeStruct((N_TOK, VPAD), jnp.float32),
        jax.ShapeDtypeStruct((NW * L,), jnp.float32),
    ],
    mesh=plsc.VectorSubcoreMesh(core_axis_name="c", subcore_axis_name="s"),
    compiler_params=pltpu.CompilerParams(
        use_tc_tiling_on_sc=True,
        needs_layout_passes=False,
    ),
    scratch_types=[
        pltpu.VMEM((VOCAB,), jnp.float32),
        pltpu.VMEM((CHUNK,), jnp.int32),
        pltpu.VMEM((CHUNK,), jnp.int32),
        pltpu.VMEM((CHUNK,), jnp.int32),
        pltpu.VMEM((CHUNK,), jnp.int32),
        pltpu.VMEM((CHUNK,), jnp.float32),
        pltpu.VMEM((CHUNK,), jnp.float32),
        pltpu.VMEM((CHUNK, VPAD), jnp.float32),
        pltpu.VMEM((CHUNK, VPAD), jnp.float32),
        pltpu.VMEM((L,), jnp.float32),
        pltpu.SemaphoreType.DMA,
        pltpu.SemaphoreType.DMA,
        pltpu.SemaphoreType.DMA,
        pltpu.SemaphoreType.DMA,
        pltpu.SemaphoreType.DMA,
        pltpu.SemaphoreType.DMA,
    ],
)
def _sc_gather_loss(idx_hbm, tgt_hbm, table_hbm, tflat_hbm, lse_hbm,
                    out_hbm, part_hbm,
                    lse_v, idx0, idx1, fl0, fl1, tv0, tv1, rows0, rows1,
                    acc_v, sg0, sg1, sw0, sw1, st0, st1):
    wid = lax.axis_index("s") * NC + lax.axis_index("c")
    base = wid * B_PER_W
    bufs = ((idx0, fl0, tv0, rows0, sg0, sw0, st0),
            (idx1, fl1, tv1, rows1, sg1, sw1, st1))

    pltpu.sync_copy(lse_hbm, lse_v)
    acc_v[...] = jnp.zeros((L,), jnp.float32)

    def load_itgt(c, idx_b, fl_b, tv_b, st_b):
        off = base + c * CHUNK
        pltpu.sync_copy(idx_hbm.at[pl.ds(off, CHUNK)], idx_b)
        pltpu.sync_copy(tgt_hbm.at[pl.ds(off, CHUNK)], fl_b)
        for j in range(CHUNK // L):
            s = pl.ds(j * L, L)
            fl_b[s] = fl_b[s] + idx_b[s] * VPAD
        pltpu.async_copy(tflat_hbm.at[fl_b], tv_b, st_b)

    def loss(idx_b, fl_b, tv_b, st_b):
        pltpu.make_async_copy(tflat_hbm.at[fl_b], tv_b, st_b).wait()
        for j in range(CHUNK // L):
            s = pl.ds(j * L, L)
            lse_g = plsc.load_gather(lse_v, [idx_b[s]])
            acc_v[...] = acc_v[...] + (lse_g - tv_b[s])

    def visit(c, b, first):
        idx_b, fl_b, tv_b, rows_b, sg_b, sw_b, st_b = bufs[b]
        idx_o, fl_o, tv_o, rows_o, sg_o, sw_o, st_o = bufs[1 - b]
        if not first:
            pltpu.make_async_copy(
                rows_o, out_hbm.at[pl.ds(base, CHUNK)], sw_o).wait()

        @pl.when(c + 1 < N_CHUNKS)
        def _():
            pltpu.async_copy(table_hbm.at[idx_o], rows_o, sg_o)

        pltpu.make_async_copy(table_hbm.at[idx_b], rows_b, sg_b).wait()
        loss(idx_b, fl_b, tv_b, st_b)
        pltpu.async_copy(
            rows_b, out_hbm.at[pl.ds(base + c * CHUNK, CHUNK)], sw_b)

        @pl.when(c + 2 < N_CHUNKS)
        def _():
            load_itgt(c + 2, idx_b, fl_b, tv_b, st_b)

    load_itgt(0, idx0, fl0, tv0, st0)
    load_itgt(1, idx1, fl1, tv1, st1)
    pltpu.async_copy(table_hbm.at[idx0], rows0, sg0)
    visit(jnp.int32(0), 0, True)
    visit(jnp.int32(1), 1, False)

    def pair(p, carry):
        visit(2 * p, 0, False)
        visit(2 * p + 1, 1, False)
        return carry

    lax.fori_loop(1, N_CHUNKS // 2, pair, 0)
    pltpu.make_async_copy(
        rows1, out_hbm.at[pl.ds(base, CHUNK)], sw1).wait()
    pltpu.sync_copy(acc_v, part_hbm.at[pl.ds(wid * L, L)])


def kernel(idx, targets, table):
    idx_f = idx.reshape(N_TOK).astype(jnp.int32)
    tgt_f = targets.reshape(N_TOK).astype(jnp.int32)
    tbl_pad = jnp.pad(table, ((0, 0), (0, VPAD - VOCAB)))
    tbl_flat = tbl_pad.reshape(VOCAB * VPAD)
    lse = _row_lse(table)
    out_pad, parts = _sc_gather_loss(idx_f, tgt_f, tbl_pad, tbl_flat, lse)
    logits2 = out_pad[:, :VOCAB]
    loss = jnp.sum(parts) / jnp.float32(N_TOK)
    return (logits2, loss)
